# Initial kernel scaffold; baseline (speedup 1.0000x reference)
import jax
import jax.numpy as jnp
from jax.experimental import pallas as pl


def kernel(simmat, dlens, dtoks, qtoks):
    raise NotImplementedError("write your pallas kernel here")


# speedup vs baseline: 23.0180x; 23.0180x over previous
import functools

import jax
import jax.numpy as jnp
from jax import lax
from jax.experimental import pallas as pl
from jax.experimental.pallas import tpu as pltpu
from jax.experimental.pallas import tpu_sc as plsc

_NBINS = 29
_NC, _NS, _L = 2, 16, 16
_NW = _NC * _NS
_STRIDE = 33
_HSIZE = 528


@functools.partial(jax.jit, static_argnums=(2, 3))
def _hist_call(sim2, dtok, c_per_b, d):
    rows = sim2.shape[0]
    rpw = rows // _NW
    chunks = d // _L

    mesh = plsc.VectorSubcoreMesh(core_axis_name="c", subcore_axis_name="s")

    @functools.partial(
        pl.kernel,
        mesh=mesh,
        compiler_params=pltpu.CompilerParams(needs_layout_passes=False),
        out_type=jax.ShapeDtypeStruct((rows, 32), jnp.float32),
        scratch_types=[
            pltpu.VMEM((rpw, d), jnp.float32),
            pltpu.VMEM((d,), jnp.int32),
            pltpu.VMEM((d,), jnp.float32),
            pltpu.VMEM((_HSIZE,), jnp.float32),
            pltpu.VMEM((rpw, 32), jnp.float32),
            pltpu.SemaphoreType.DMA,
        ],
    )
    def body(sim_hbm, dtok_hbm, out_hbm, sim_v, dtok_v, wd_v, hist_v, out_v, sem):
        wid = lax.axis_index("s") * _NC + lax.axis_index("c")
        base = wid * rpw
        b = base // c_per_b

        cp = pltpu.async_copy(sim_hbm.at[pl.ds(base, rpw)], sim_v, sem)
        pltpu.sync_copy(dtok_hbm.at[b], dtok_v)

        zero = jnp.zeros((_L,), jnp.float32)
        one = jnp.ones((_L,), jnp.float32)

        def zbody(i, carry):
            hist_v[pl.ds(i * _L, _L)] = zero
            return carry

        lax.fori_loop(0, _HSIZE // _L, zbody, 0)

        def wbody(i, carry):
            t = dtok_v[pl.ds(i * _L, _L)]
            wd_v[pl.ds(i * _L, _L)] = jnp.where(t != jnp.int32(-1), one, zero)
            return carry

        lax.fori_loop(0, chunks, wbody, 0)

        cp.wait()

        lane_off = lax.iota(jnp.int32, _L) * _STRIDE

        def row_body(r, carry):
            def cbody(ci, carry2):
                s = sim_v[r, pl.ds(ci * _L, _L)]
                bins = (((s + 1.00001) / 2.0) * (_NBINS - 1)).astype(jnp.int32)
                w = wd_v[pl.ds(ci * _L, _L)]
                plsc.addupdate_scatter(hist_v, [bins + lane_off], w)
                return carry2

            lax.fori_loop(0, chunks, cbody, 0)

            acc0 = jnp.zeros((_L,), jnp.float32)
            acc1 = jnp.zeros((_L,), jnp.float32)
            for lane in range(_L):
                acc0 = acc0 + hist_v[pl.ds(lane * _STRIDE, _L)]
                acc1 = acc1 + hist_v[pl.ds(lane * _STRIDE + _L, _L)]
                hist_v[pl.ds(lane * _STRIDE, _L)] = zero
                hist_v[pl.ds(lane * _STRIDE + _L, _L)] = zero
            out_v[r, pl.ds(0, _L)] = acc0
            out_v[r, pl.ds(_L, _L)] = acc1
            return carry

        lax.fori_loop(0, rpw, row_body, 0)

        pltpu.sync_copy(out_v, out_hbm.at[pl.ds(base, rpw)])

    return body(sim2, dtok)


def kernel(simmat, dlens, dtoks, qtoks):
    del dlens
    B, C, Q, D = simmat.shape
    sim2 = simmat.reshape(B * C * Q, D)
    dtok = dtoks.astype(jnp.int32)
    out = _hist_call(sim2, dtok, C * Q, D)
    hist = out[:, :_NBINS].reshape(B, C, Q, _NBINS)
    mq = (qtoks != -1).astype(jnp.float32)
    return hist * mq[:, None, :, None]


# speedup vs baseline: 25.2629x; 1.0975x over previous
import functools

import jax
import jax.numpy as jnp
from jax import lax
from jax.experimental import pallas as pl
from jax.experimental.pallas import tpu as pltpu
from jax.experimental.pallas import tpu_sc as plsc

_NBINS = 29
_NC, _NS, _L = 2, 16, 16
_NW = _NC * _NS
_STRIDE = 33
_REGION = 16 * _STRIDE
_NREG = 2
_UNROLL = 4


@functools.partial(jax.jit, static_argnums=(2, 3))
def _hist_call(sim2, dtok, c_per_b, d):
    rows = sim2.shape[0]
    rpw = rows // _NW
    half = rpw // 2
    chunks = d // _L
    steps = chunks // _UNROLL

    mesh = plsc.VectorSubcoreMesh(core_axis_name="c", subcore_axis_name="s")

    @functools.partial(
        pl.kernel,
        mesh=mesh,
        compiler_params=pltpu.CompilerParams(needs_layout_passes=False),
        out_type=jax.ShapeDtypeStruct((rows, 32), jnp.float32),
        scratch_types=[
            pltpu.VMEM((rpw, d), jnp.float32),
            pltpu.VMEM((d,), jnp.int32),
            pltpu.VMEM((d,), jnp.float32),
            pltpu.VMEM((_NREG * _REGION,), jnp.float32),
            pltpu.VMEM((rpw, 32), jnp.float32),
            pltpu.SemaphoreType.DMA,
            pltpu.SemaphoreType.DMA,
        ],
    )
    def body(sim_hbm, dtok_hbm, out_hbm, sim_v, dtok_v, wd_v, hist_v, out_v,
             sem0, sem1):
        wid = lax.axis_index("s") * _NC + lax.axis_index("c")
        base = wid * rpw
        b = base // c_per_b

        cp0 = pltpu.async_copy(
            sim_hbm.at[pl.ds(base, half)], sim_v.at[pl.ds(0, half)], sem0)
        cp1 = pltpu.async_copy(
            sim_hbm.at[pl.ds(base + half, half)], sim_v.at[pl.ds(half, half)],
            sem1)
        pltpu.sync_copy(dtok_hbm.at[b], dtok_v)

        zero = jnp.zeros((_L,), jnp.float32)
        one = jnp.ones((_L,), jnp.float32)

        def zbody(i, carry):
            hist_v[pl.ds(i * _L, _L)] = zero
            return carry

        lax.fori_loop(0, _NREG * _REGION // _L, zbody, 0)

        def wbody(i, carry):
            for j in range(_UNROLL):
                off = i * (_UNROLL * _L) + j * _L
                t = dtok_v[pl.ds(off, _L)]
                wd_v[pl.ds(off, _L)] = jnp.where(t != jnp.int32(-1), one, zero)
            return carry

        lax.fori_loop(0, steps, wbody, 0)

        lane = lax.iota(jnp.int32, _L) * _STRIDE
        lane_off = [lane + (j % _NREG) * _REGION for j in range(_UNROLL)]

        def row_body(r, carry):
            def cbody(ci, carry2):
                for j in range(_UNROLL):
                    off = ci * (_UNROLL * _L) + j * _L
                    s = sim_v[r, pl.ds(off, _L)]
                    bins = ((s + 1.00001) * 14.0).astype(jnp.int32)
                    w = wd_v[pl.ds(off, _L)]
                    plsc.addupdate_scatter(hist_v, [bins + lane_off[j]], w)
                return carry2

            lax.fori_loop(0, steps, cbody, 0)

            acc0 = jnp.zeros((_L,), jnp.float32)
            acc1 = jnp.zeros((_L,), jnp.float32)
            for g in range(_NREG):
                for ln in range(_L):
                    o = g * _REGION + ln * _STRIDE
                    acc0 = acc0 + hist_v[pl.ds(o, _L)]
                    acc1 = acc1 + hist_v[pl.ds(o + _L, _L)]
                    hist_v[pl.ds(o, _L)] = zero
                    hist_v[pl.ds(o + _L, _L)] = zero
            out_v[r, pl.ds(0, _L)] = acc0
            out_v[r, pl.ds(_L, _L)] = acc1
            return carry

        cp0.wait()
        lax.fori_loop(0, half, row_body, 0)
        cp1.wait()
        lax.fori_loop(half, rpw, row_body, 0)

        pltpu.sync_copy(out_v, out_hbm.at[pl.ds(base, rpw)])

    return body(sim2, dtok)


def kernel(simmat, dlens, dtoks, qtoks):
    del dlens
    B, C, Q, D = simmat.shape
    sim2 = simmat.reshape(B * C * Q, D)
    dtok = dtoks.astype(jnp.int32)
    out = _hist_call(sim2, dtok, C * Q, D)
    hist = out[:, :_NBINS].reshape(B, C, Q, _NBINS)
    mq = (qtoks != -1).astype(jnp.float32)
    return hist * mq[:, None, :, None]


# speedup vs baseline: 48.8936x; 1.9354x over previous
import functools

import jax
import jax.numpy as jnp
from jax import lax
from jax.experimental import pallas as pl
from jax.experimental.pallas import tpu as pltpu
from jax.experimental.pallas import tpu_sc as plsc

_NBINS = 29
_NC, _NS, _L = 2, 16, 16
_NW = _NC * _NS
_STRIDE = 33
_REGION = 16 * _STRIDE
_NREG = 4
_UNROLL = 8


@functools.partial(jax.jit, static_argnums=(2, 3))
def _hist_call(sim2, dtok, c_per_b, d):
    rows = sim2.shape[0]
    rpw = rows // _NW
    half = rpw // 2
    chunks = d // _L
    steps = chunks // _UNROLL

    mesh = plsc.VectorSubcoreMesh(core_axis_name="c", subcore_axis_name="s")

    @functools.partial(
        pl.kernel,
        mesh=mesh,
        compiler_params=pltpu.CompilerParams(needs_layout_passes=False),
        out_type=jax.ShapeDtypeStruct((rows, 32), jnp.float32),
        scratch_types=[
            pltpu.VMEM((rpw, d), jnp.float32),
            pltpu.VMEM((d,), jnp.int32),
            pltpu.VMEM((d,), jnp.float32),
            pltpu.VMEM((_NREG * _REGION,), jnp.float32),
            pltpu.VMEM((rpw, 32), jnp.float32),
            pltpu.SemaphoreType.DMA,
            pltpu.SemaphoreType.DMA,
        ],
    )
    def body(sim_hbm, dtok_hbm, out_hbm, sim_v, dtok_v, wd_v, hist_v, out_v,
             sem0, sem1):
        wid = lax.axis_index("s") * _NC + lax.axis_index("c")
        base = wid * rpw
        b = base // c_per_b

        cp0 = pltpu.async_copy(
            sim_hbm.at[pl.ds(base, half)], sim_v.at[pl.ds(0, half)], sem0)
        cp1 = pltpu.async_copy(
            sim_hbm.at[pl.ds(base + half, half)], sim_v.at[pl.ds(half, half)],
            sem1)
        pltpu.sync_copy(dtok_hbm.at[b], dtok_v)

        zero = jnp.zeros((_L,), jnp.float32)
        one = jnp.ones((_L,), jnp.float32)

        @plsc.parallel_loop(0, _NREG * _REGION // _L, unroll=4)
        def _zbody(i):
            hist_v[pl.ds(i * _L, _L)] = zero

        @plsc.parallel_loop(0, chunks, unroll=4)
        def _wbody(i):
            t = dtok_v[pl.ds(i * _L, _L)]
            wd_v[pl.ds(i * _L, _L)] = jnp.where(t != jnp.int32(-1), one, zero)

        lane = lax.iota(jnp.int32, _L) * _STRIDE

        def row_body(r, carry):
            @plsc.parallel_loop(0, chunks, unroll=_UNROLL)
            def _cbody(ci):
                s = sim_v[r, pl.ds(ci * _L, _L)]
                bins = ((s + 1.00001) * 14.0).astype(jnp.int32)
                w = wd_v[pl.ds(ci * _L, _L)]
                reg = (ci & (_NREG - 1)) * _REGION
                plsc.addupdate_scatter(hist_v, [bins + lane + reg], w)

            acc0 = jnp.zeros((_L,), jnp.float32)
            acc1 = jnp.zeros((_L,), jnp.float32)
            for g in range(_NREG):
                for ln in range(_L):
                    o = g * _REGION + ln * _STRIDE
                    acc0 = acc0 + hist_v[pl.ds(o, _L)]
                    acc1 = acc1 + hist_v[pl.ds(o + _L, _L)]
                    hist_v[pl.ds(o, _L)] = zero
                    hist_v[pl.ds(o + _L, _L)] = zero
            out_v[r, pl.ds(0, _L)] = acc0
            out_v[r, pl.ds(_L, _L)] = acc1
            return carry

        cp0.wait()
        lax.fori_loop(0, half, row_body, 0)
        cp1.wait()
        lax.fori_loop(half, rpw, row_body, 0)

        pltpu.sync_copy(out_v, out_hbm.at[pl.ds(base, rpw)])

    return body(sim2, dtok)


def kernel(simmat, dlens, dtoks, qtoks):
    del dlens
    B, C, Q, D = simmat.shape
    sim2 = simmat.reshape(B * C * Q, D)
    dtok = dtoks.astype(jnp.int32)
    out = _hist_call(sim2, dtok, C * Q, D)
    hist = out[:, :_NBINS].reshape(B, C, Q, _NBINS)
    mq = (qtoks != -1).astype(jnp.float32)
    return hist * mq[:, None, :, None]


# speedup vs baseline: 50.8150x; 1.0393x over previous
import functools

import jax
import jax.numpy as jnp
from jax import lax
from jax.experimental import pallas as pl
from jax.experimental.pallas import tpu as pltpu
from jax.experimental.pallas import tpu_sc as plsc

_NBINS = 29
_NC, _NS, _L = 2, 16, 16
_NW = _NC * _NS
_STRIDE = 33
_REGION = 16 * _STRIDE + 1
_NREG = 2
_UNROLL = 8
_HSIZE = 1072


@functools.partial(jax.jit, static_argnums=(2, 3))
def _hist_call(sim2, dtok, c_per_b, d):
    rows = sim2.shape[0]
    rpw = rows // _NW
    half = rpw // 2
    chunks = d // _L
    steps = chunks // _UNROLL

    mesh = plsc.VectorSubcoreMesh(core_axis_name="c", subcore_axis_name="s")

    @functools.partial(
        pl.kernel,
        mesh=mesh,
        compiler_params=pltpu.CompilerParams(needs_layout_passes=False),
        out_type=jax.ShapeDtypeStruct((rows, 32), jnp.float32),
        scratch_types=[
            pltpu.VMEM((rpw, d), jnp.float32),
            pltpu.VMEM((d,), jnp.int32),
            pltpu.VMEM((d,), jnp.float32),
            pltpu.VMEM((_HSIZE,), jnp.float32),
            pltpu.VMEM((rpw, 32), jnp.float32),
            pltpu.SemaphoreType.DMA,
            pltpu.SemaphoreType.DMA,
        ],
    )
    def body(sim_hbm, dtok_hbm, out_hbm, sim_v, dtok_v, wd_v, hist_v, out_v,
             sem0, sem1):
        wid = lax.axis_index("s") * _NC + lax.axis_index("c")
        base = wid * rpw
        b = base // c_per_b

        cp0 = pltpu.async_copy(
            sim_hbm.at[pl.ds(base, half)], sim_v.at[pl.ds(0, half)], sem0)
        cp1 = pltpu.async_copy(
            sim_hbm.at[pl.ds(base + half, half)], sim_v.at[pl.ds(half, half)],
            sem1)
        pltpu.sync_copy(dtok_hbm.at[b], dtok_v)

        zero = jnp.zeros((_L,), jnp.float32)
        one = jnp.ones((_L,), jnp.float32)

        @plsc.parallel_loop(0, _HSIZE // _L, unroll=4)
        def _zbody(i):
            hist_v[pl.ds(i * _L, _L)] = zero

        @plsc.parallel_loop(0, chunks, unroll=4)
        def _wbody(i):
            t = dtok_v[pl.ds(i * _L, _L)]
            wd_v[pl.ds(i * _L, _L)] = jnp.where(t != jnp.int32(-1), one, zero)

        lane = lax.iota(jnp.int32, _L) * _STRIDE

        def row_body(r, carry):
            @plsc.parallel_loop(0, chunks, unroll=_UNROLL)
            def _cbody(ci):
                s = sim_v[r, pl.ds(ci * _L, _L)]
                bins = ((s + 1.00001) * 14.0).astype(jnp.int32)
                w = wd_v[pl.ds(ci * _L, _L)]
                reg = (ci & (_NREG - 1)) * _REGION
                plsc.addupdate_scatter(hist_v, [bins + lane + reg], w)

            @plsc.parallel_loop(
                0, _L, unroll=2,
                carry=(jnp.zeros((_L,), jnp.float32),
                       jnp.zeros((_L,), jnp.float32)))
            def _accs(ln, accs):
                a0, a1 = accs
                o = ln * _STRIDE
                t0 = hist_v[pl.ds(o, _L)] + hist_v[pl.ds(o + _REGION, _L)]
                t1 = (hist_v[pl.ds(o + _L, _L)]
                      + hist_v[pl.ds(o + _REGION + _L, _L)])
                hist_v[pl.ds(o, _L)] = zero
                hist_v[pl.ds(o + _REGION, _L)] = zero
                hist_v[pl.ds(o + _L, _L)] = zero
                hist_v[pl.ds(o + _REGION + _L, _L)] = zero
                return (a0 + t0, a1 + t1)

            acc0, acc1 = _accs
            out_v[r, pl.ds(0, _L)] = acc0
            out_v[r, pl.ds(_L, _L)] = acc1
            return carry

        cp0.wait()
        lax.fori_loop(0, half, row_body, 0)
        cp1.wait()
        lax.fori_loop(half, rpw, row_body, 0)

        pltpu.sync_copy(out_v, out_hbm.at[pl.ds(base, rpw)])

    return body(sim2, dtok)


def kernel(simmat, dlens, dtoks, qtoks):
    del dlens
    B, C, Q, D = simmat.shape
    sim2 = simmat.reshape(B * C * Q, D)
    dtok = dtoks.astype(jnp.int32)
    out = _hist_call(sim2, dtok, C * Q, D)
    hist = out[:, :_NBINS].reshape(B, C, Q, _NBINS)
    mq = (qtoks != -1).astype(jnp.float32)
    return hist * mq[:, None, :, None]


# speedup vs baseline: 54.5310x; 1.0731x over previous
import functools

import jax
import jax.numpy as jnp
from jax import lax
from jax.experimental import pallas as pl
from jax.experimental.pallas import tpu as pltpu
from jax.experimental.pallas import tpu_sc as plsc

_NBINS = 29
_NC, _NS, _L = 2, 16, 16
_NW = _NC * _NS
_LSTR = 33
_RSTR = 545


@functools.partial(jax.jit, static_argnums=(2, 3))
def _hist_call(sim2, dtok, c_per_b, d):
    rows = sim2.shape[0]
    rpw = rows // _NW
    chunks = d // _L
    hsize = rpw * _RSTR

    mesh = plsc.VectorSubcoreMesh(core_axis_name="c", subcore_axis_name="s")

    @functools.partial(
        pl.kernel,
        mesh=mesh,
        compiler_params=pltpu.CompilerParams(needs_layout_passes=False),
        out_type=jax.ShapeDtypeStruct((rows, 32), jnp.float32),
        scratch_types=[
            pltpu.VMEM((rpw, d), jnp.float32),
            pltpu.VMEM((d,), jnp.int32),
            pltpu.VMEM((d,), jnp.float32),
            pltpu.VMEM((hsize,), jnp.float32),
            pltpu.VMEM((rpw, 32), jnp.float32),
            pltpu.SemaphoreType.DMA,
        ],
    )
    def body(sim_hbm, dtok_hbm, out_hbm, sim_v, dtok_v, wd_v, hist_v, out_v,
             sem):
        wid = lax.axis_index("s") * _NC + lax.axis_index("c")
        base = wid * rpw
        b = base // c_per_b

        cp = pltpu.async_copy(sim_hbm.at[pl.ds(base, rpw)], sim_v, sem)
        pltpu.sync_copy(dtok_hbm.at[b], dtok_v)

        zero = jnp.zeros((_L,), jnp.float32)
        one = jnp.ones((_L,), jnp.float32)

        @plsc.parallel_loop(0, hsize // _L, unroll=4)
        def _zbody(i):
            hist_v[pl.ds(i * _L, _L)] = zero

        @plsc.parallel_loop(0, chunks, unroll=4)
        def _wbody(i):
            t = dtok_v[pl.ds(i * _L, _L)]
            wd_v[pl.ds(i * _L, _L)] = jnp.where(t != jnp.int32(-1), one, zero)

        cp.wait()

        lane = lax.iota(jnp.int32, _L) * _LSTR

        def cbody(ci, carry):
            w = wd_v[pl.ds(ci * _L, _L)]

            @plsc.parallel_loop(0, rpw, unroll=4)
            def _rbody(r):
                s = sim_v[r, pl.ds(ci * _L, _L)]
                bins = ((s + 1.00001) * 14.0).astype(jnp.int32)
                plsc.addupdate_scatter(hist_v, [bins + lane + r * _RSTR], w)

            return carry

        lax.fori_loop(0, chunks, cbody, 0)

        def red_body(r, carry):
            @plsc.parallel_loop(
                0, _L, unroll=2,
                carry=(jnp.zeros((_L,), jnp.float32),
                       jnp.zeros((_L,), jnp.float32)))
            def _accs(ln, accs):
                a0, a1 = accs
                o = r * _RSTR + ln * _LSTR
                t0 = hist_v[pl.ds(o, _L)]
                t1 = hist_v[pl.ds(o + _L, _L)]
                hist_v[pl.ds(o, _L)] = zero
                hist_v[pl.ds(o + _L, _L)] = zero
                return (a0 + t0, a1 + t1)

            acc0, acc1 = _accs
            out_v[r, pl.ds(0, _L)] = acc0
            out_v[r, pl.ds(_L, _L)] = acc1
            return carry

        lax.fori_loop(0, rpw, red_body, 0)

        pltpu.sync_copy(out_v, out_hbm.at[pl.ds(base, rpw)])

    return body(sim2, dtok)


def kernel(simmat, dlens, dtoks, qtoks):
    del dlens
    B, C, Q, D = simmat.shape
    sim2 = simmat.reshape(B * C * Q, D)
    dtok = dtoks.astype(jnp.int32)
    out = _hist_call(sim2, dtok, C * Q, D)
    hist = out[:, :_NBINS].reshape(B, C, Q, _NBINS)
    mq = (qtoks != -1).astype(jnp.float32)
    return hist * mq[:, None, :, None]


# speedup vs baseline: 54.9986x; 1.0086x over previous
import functools

import jax
import jax.numpy as jnp
from jax import lax
from jax.experimental import pallas as pl
from jax.experimental.pallas import tpu as pltpu
from jax.experimental.pallas import tpu_sc as plsc

_NBINS = 29
_NC, _NS, _L = 2, 16, 16
_NW = _NC * _NS
_LSTR = 33
_RSTR = 545
_QUAD = 4


@functools.partial(jax.jit, static_argnums=(3, 4))
def _hist_call(sim2, dtok, qtok, c_per_b, d):
    rows = sim2.shape[0]
    rpw = rows // _NW
    chunks = d // _L
    hsize = rpw * _RSTR

    mesh = plsc.VectorSubcoreMesh(core_axis_name="c", subcore_axis_name="s")

    @functools.partial(
        pl.kernel,
        mesh=mesh,
        compiler_params=pltpu.CompilerParams(needs_layout_passes=False),
        out_type=jax.ShapeDtypeStruct((rows, 32), jnp.float32),
        scratch_types=[
            pltpu.VMEM((rpw, d), jnp.float32),
            pltpu.VMEM((d,), jnp.int32),
            pltpu.VMEM((_L,), jnp.int32),
            pltpu.VMEM((d,), jnp.float32),
            pltpu.VMEM((_L,), jnp.float32),
            pltpu.VMEM((hsize,), jnp.float32),
            pltpu.VMEM((rpw, 32), jnp.float32),
            pltpu.SemaphoreType.DMA,
            pltpu.SemaphoreType.DMA,
        ],
    )
    def body(sim_hbm, dtok_hbm, qtok_hbm, out_hbm, sim_v, dtok_v, qtok_v,
             wd_v, mq_v, hist_v, out_v, sem0, sem1):
        wid = lax.axis_index("s") * _NC + lax.axis_index("c")
        base = wid * rpw
        b = base // c_per_b

        cp0 = pltpu.async_copy(
            sim_hbm.at[pl.ds(base, rpw), pl.ds(0, d // 2)],
            sim_v.at[:, pl.ds(0, d // 2)], sem0)
        cp1 = pltpu.async_copy(
            sim_hbm.at[pl.ds(base, rpw), pl.ds(d // 2, d // 2)],
            sim_v.at[:, pl.ds(d // 2, d // 2)], sem1)
        pltpu.sync_copy(dtok_hbm.at[b], dtok_v)
        pltpu.sync_copy(qtok_hbm.at[b], qtok_v)

        zero = jnp.zeros((_L,), jnp.float32)
        one = jnp.ones((_L,), jnp.float32)

        @plsc.parallel_loop(0, hsize // _L, unroll=4)
        def _zbody(i):
            hist_v[pl.ds(i * _L, _L)] = zero

        @plsc.parallel_loop(0, chunks, unroll=4)
        def _wbody(i):
            t = dtok_v[pl.ds(i * _L, _L)]
            wd_v[pl.ds(i * _L, _L)] = jnp.where(t != jnp.int32(-1), one, zero)

        qt = qtok_v[pl.ds(0, _L)]
        mq_v[pl.ds(0, _L)] = jnp.where(qt != jnp.int32(-1), one, zero)

        lane = lax.iota(jnp.int32, _L) * _LSTR

        def quad_body(cq, carry):
            cb = cq * (_QUAD * _L)
            ws = [wd_v[pl.ds(cb + j * _L, _L)] for j in range(_QUAD)]

            @plsc.parallel_loop(0, rpw, unroll=2)
            def _rbody(r):
                lane_r = lane + r * _RSTR
                for j in range(_QUAD):
                    s = sim_v[r, pl.ds(cb + j * _L, _L)]
                    bins = ((s + 1.00001) * 14.0).astype(jnp.int32)
                    plsc.addupdate_scatter(hist_v, [bins + lane_r], ws[j])

            return carry

        cp0.wait()
        lax.fori_loop(0, chunks // (2 * _QUAD), quad_body, 0)
        cp1.wait()
        lax.fori_loop(chunks // (2 * _QUAD), chunks // _QUAD, quad_body, 0)

        def red_body(r, carry):
            @plsc.parallel_loop(
                0, _L, unroll=2,
                carry=(jnp.zeros((_L,), jnp.float32),
                       jnp.zeros((_L,), jnp.float32)))
            def _accs(ln, accs):
                a0, a1 = accs
                o = r * _RSTR + ln * _LSTR
                t0 = hist_v[pl.ds(o, _L)]
                t1 = hist_v[pl.ds(o + _L, _L)]
                hist_v[pl.ds(o, _L)] = zero
                hist_v[pl.ds(o + _L, _L)] = zero
                return (a0 + t0, a1 + t1)

            acc0, acc1 = _accs
            mq = plsc.load_gather(mq_v, [jnp.full((_L,), r % 16, jnp.int32)])
            out_v[r, pl.ds(0, _L)] = acc0 * mq
            out_v[r, pl.ds(_L, _L)] = acc1 * mq
            return carry

        lax.fori_loop(0, rpw, red_body, 0)

        pltpu.sync_copy(out_v, out_hbm.at[pl.ds(base, rpw)])

    return body(sim2, dtok, qtok)


def kernel(simmat, dlens, dtoks, qtoks):
    del dlens
    B, C, Q, D = simmat.shape
    sim2 = simmat.reshape(B * C * Q, D)
    dtok = dtoks.astype(jnp.int32)
    qtok = qtoks.astype(jnp.int32)
    out = _hist_call(sim2, dtok, qtok, C * Q, D)
    return out[:, :_NBINS].reshape(B, C, Q, _NBINS)


# speedup vs baseline: 57.4332x; 1.0443x over previous
import functools

import jax
import jax.numpy as jnp
from jax import lax
from jax.experimental import pallas as pl
from jax.experimental.pallas import tpu as pltpu
from jax.experimental.pallas import tpu_sc as plsc

_NBINS = 29
_NC, _NS, _L = 2, 16, 16
_NW = _NC * _NS
_LSTR = 33
_RSTR = 545
_QUAD = 4


@functools.partial(jax.jit, static_argnums=(3, 4))
def _hist_call(sim2, dtok, qtok, c_per_b, d):
    rows = sim2.shape[0]
    rpw = rows // _NW
    chunks = d // _L
    hsize = rpw * _RSTR

    mesh = plsc.VectorSubcoreMesh(core_axis_name="c", subcore_axis_name="s")

    @functools.partial(
        pl.kernel,
        mesh=mesh,
        compiler_params=pltpu.CompilerParams(needs_layout_passes=False),
        out_type=jax.ShapeDtypeStruct((rows, 32), jnp.float32),
        scratch_types=[
            pltpu.VMEM((rpw, d), jnp.float32),
            pltpu.VMEM((d,), jnp.int32),
            pltpu.VMEM((_L,), jnp.int32),
            pltpu.VMEM((d,), jnp.float32),
            pltpu.VMEM((_L,), jnp.float32),
            pltpu.VMEM((hsize,), jnp.float32),
            pltpu.VMEM((rpw, 32), jnp.float32),
            pltpu.SemaphoreType.DMA,
            pltpu.SemaphoreType.DMA,
        ],
    )
    def body(sim_hbm, dtok_hbm, qtok_hbm, out_hbm, sim_v, dtok_v, qtok_v,
             wd_v, mq_v, hist_v, out_v, sem0, sem1):
        wid = lax.axis_index("s") * _NC + lax.axis_index("c")
        base = wid * rpw
        b = base // c_per_b

        cp0 = pltpu.async_copy(sim_hbm.at[pl.ds(base, rpw)], sim_v, sem0)
        pltpu.sync_copy(dtok_hbm.at[b], dtok_v)
        pltpu.sync_copy(qtok_hbm.at[b], qtok_v)

        zero = jnp.zeros((_L,), jnp.float32)
        one = jnp.ones((_L,), jnp.float32)

        @plsc.parallel_loop(0, hsize // _L, unroll=4)
        def _zbody(i):
            hist_v[pl.ds(i * _L, _L)] = zero

        @plsc.parallel_loop(0, chunks, unroll=4)
        def _wbody(i):
            t = dtok_v[pl.ds(i * _L, _L)]
            wd_v[pl.ds(i * _L, _L)] = jnp.where(t != jnp.int32(-1), one, zero)

        qt = qtok_v[pl.ds(0, _L)]
        mq_v[pl.ds(0, _L)] = jnp.where(qt != jnp.int32(-1), one, zero)

        lane = lax.iota(jnp.int32, _L) * _LSTR

        def quad_body(cq, carry):
            cb = cq * (_QUAD * _L)
            ws = [wd_v[pl.ds(cb + j * _L, _L)] for j in range(_QUAD)]

            @plsc.parallel_loop(0, rpw, unroll=2)
            def _rbody(r):
                lane_r = lane + r * _RSTR
                for j in range(_QUAD):
                    s = sim_v[r, pl.ds(cb + j * _L, _L)]
                    bins = ((s + 1.00001) * 14.0).astype(jnp.int32)
                    plsc.addupdate_scatter(hist_v, [bins + lane_r], ws[j])

            return carry

        cp0.wait()
        lax.fori_loop(0, chunks // _QUAD, quad_body, 0)

        def red_body(r, carry):
            @plsc.parallel_loop(
                0, _L, unroll=2,
                carry=(jnp.zeros((_L,), jnp.float32),
                       jnp.zeros((_L,), jnp.float32)))
            def _accs(ln, accs):
                a0, a1 = accs
                o = r * _RSTR + ln * _LSTR
                t0 = hist_v[pl.ds(o, _L)]
                t1 = hist_v[pl.ds(o + _L, _L)]
                hist_v[pl.ds(o, _L)] = zero
                hist_v[pl.ds(o + _L, _L)] = zero
                return (a0 + t0, a1 + t1)

            acc0, acc1 = _accs
            mq = plsc.load_gather(mq_v, [jnp.full((_L,), r % 16, jnp.int32)])
            out_v[r, pl.ds(0, _L)] = acc0 * mq
            out_v[r, pl.ds(_L, _L)] = acc1 * mq
            return carry

        lax.fori_loop(0, rpw, red_body, 0)

        pltpu.sync_copy(out_v, out_hbm.at[pl.ds(base, rpw)])

    return body(sim2, dtok, qtok)


def kernel(simmat, dlens, dtoks, qtoks):
    del dlens
    B, C, Q, D = simmat.shape
    sim2 = simmat.reshape(B * C * Q, D)
    dtok = dtoks.astype(jnp.int32)
    qtok = qtoks.astype(jnp.int32)
    out = _hist_call(sim2, dtok, qtok, C * Q, D)
    return out[:, :_NBINS].reshape(B, C, Q, _NBINS)
